# baseline (device time: 21103 ns/iter reference)
import os

import jax
import jax.numpy as jnp
from jax import lax
from jax.experimental import pallas as pl
from jax.experimental.pallas import tpu as pltpu

K = 16


def kernel(x):
    m_per, n = x.shape
    half = m_per // 2
    ch = half // K

    def body(x_ref, out_ref, stage_ref, x_send, x_recv, y_send, y_recv):
        my_x = lax.axis_index("x")
        my_y = lax.axis_index("y")
        my_z = lax.axis_index("z")
        xpeer = (1 - my_x, my_y, my_z)
        ypeer = (my_x, 1 - my_y, my_z)

        stage_ref[...] = x_ref[pl.ds(my_y * half, half), :].astype(jnp.bfloat16)

        barrier_sem = pltpu.get_barrier_semaphore()
        for nbr in (xpeer, ypeer):
            pl.semaphore_signal(
                barrier_sem, inc=1, device_id=nbr,
                device_id_type=pl.DeviceIdType.MESH,
            )
        pl.semaphore_wait(barrier_sem, 2)

        x_base = my_x * m_per + my_y * half
        r_base = (1 - my_x) * m_per + my_y * half

        x_rdmas = []
        for k in range(K):
            r = pltpu.make_async_remote_copy(
                src_ref=stage_ref.at[pl.ds(k * ch, ch), :],
                dst_ref=out_ref.at[pl.ds(x_base + k * ch, ch), :],
                send_sem=x_send.at[k],
                recv_sem=x_recv.at[k],
                device_id=xpeer,
                device_id_type=pl.DeviceIdType.MESH,
            )
            r.start()
            x_rdmas.append(r)

        out_ref[pl.ds(my_x * m_per, m_per), :] = x_ref[...].astype(jnp.bfloat16)

        y_rdmas = []
        for k in range(K):
            x_rdmas[k].wait_recv()
            r = pltpu.make_async_remote_copy(
                src_ref=out_ref.at[pl.ds(r_base + k * ch, ch), :],
                dst_ref=out_ref.at[pl.ds(r_base + k * ch, ch), :],
                send_sem=y_send.at[k],
                recv_sem=y_recv.at[k],
                device_id=ypeer,
                device_id_type=pl.DeviceIdType.MESH,
            )
            r.start()
            y_rdmas.append(r)

        for k in range(K):
            x_rdmas[k].wait_send()
            y_rdmas[k].wait_send()
            y_rdmas[k].wait_recv()

    return pl.pallas_call(
        body,
        out_shape=jax.ShapeDtypeStruct((2 * m_per, n), jnp.bfloat16),
        in_specs=[pl.BlockSpec(memory_space=pltpu.VMEM)],
        out_specs=pl.BlockSpec(memory_space=pltpu.VMEM),
        scratch_shapes=[
            pltpu.VMEM((half, n), jnp.bfloat16),
            pltpu.SemaphoreType.DMA((K,)),
            pltpu.SemaphoreType.DMA((K,)),
            pltpu.SemaphoreType.DMA((K,)),
            pltpu.SemaphoreType.DMA((K,)),
        ],
        compiler_params=pltpu.CompilerParams(collective_id=0),
    )(x)


def _make_bench_kernel(variant):

    def kern(x):
        m_per, n = x.shape
        half = m_per // 2
        ch = half // K

        def body(x_ref, out_ref, stage_ref, s_send, s_recv, t_send, t_recv,
                 u_send, u_recv):
            my_x = lax.axis_index("x")
            my_y = lax.axis_index("y")
            my_z = lax.axis_index("z")
            xpeer = (1 - my_x, my_y, my_z)
            ypeer = (my_x, 1 - my_y, my_z)
            zpeer = (my_x, my_y, my_z - 2 * (my_z % 2) + 1)

            stage_ref[...] = x_ref[:half, :].astype(jnp.bfloat16)

            nbrs = {"xonly": [xpeer], "xy_indep": [xpeer, ypeer],
                    "xyz_indep": [xpeer, ypeer, zpeer]}[variant]
            barrier_sem = pltpu.get_barrier_semaphore()
            for nbr in nbrs:
                pl.semaphore_signal(
                    barrier_sem, inc=1, device_id=nbr,
                    device_id_type=pl.DeviceIdType.MESH,
                )
            pl.semaphore_wait(barrier_sem, len(nbrs))

            rdmas = []
            for k in range(K):
                r = pltpu.make_async_remote_copy(
                    src_ref=stage_ref.at[pl.ds(k * ch, ch), :],
                    dst_ref=out_ref.at[pl.ds(k * ch, ch), :],
                    send_sem=s_send.at[k], recv_sem=s_recv.at[k],
                    device_id=xpeer, device_id_type=pl.DeviceIdType.MESH,
                )
                r.start()
                rdmas.append(r)
                if variant in ("xy_indep", "xyz_indep"):
                    r2 = pltpu.make_async_remote_copy(
                        src_ref=stage_ref.at[pl.ds(k * ch, ch), :],
                        dst_ref=out_ref.at[pl.ds(half + k * ch, ch), :],
                        send_sem=t_send.at[k], recv_sem=t_recv.at[k],
                        device_id=ypeer, device_id_type=pl.DeviceIdType.MESH,
                    )
                    r2.start()
                    rdmas.append(r2)
                if variant == "xyz_indep":
                    r3 = pltpu.make_async_remote_copy(
                        src_ref=stage_ref.at[pl.ds(k * ch, ch), :],
                        dst_ref=out_ref.at[pl.ds(2 * half + k * ch, ch), :],
                        send_sem=u_send.at[k], recv_sem=u_recv.at[k],
                        device_id=zpeer, device_id_type=pl.DeviceIdType.MESH,
                    )
                    r3.start()
                    rdmas.append(r3)

            for r in rdmas:
                r.wait_send()
            for r in rdmas:
                r.wait_recv()

        return pl.pallas_call(
            body,
            out_shape=jax.ShapeDtypeStruct((2 * m_per, n), jnp.bfloat16),
            in_specs=[pl.BlockSpec(memory_space=pltpu.VMEM)],
            out_specs=pl.BlockSpec(memory_space=pltpu.VMEM),
            scratch_shapes=[
                pltpu.VMEM((half, n), jnp.bfloat16),
                pltpu.SemaphoreType.DMA((K,)),
                pltpu.SemaphoreType.DMA((K,)),
                pltpu.SemaphoreType.DMA((K,)),
                pltpu.SemaphoreType.DMA((K,)),
                pltpu.SemaphoreType.DMA((K,)),
                pltpu.SemaphoreType.DMA((K,)),
            ],
            compiler_params=pltpu.CompilerParams(collective_id=0),
        )(x)

    return kern


_BENCH = os.environ.get("BENCH_VARIANT")
if _BENCH:
    kernel = _make_bench_kernel(_BENCH)


# device time: 20771 ns/iter; 1.0160x vs baseline; 1.0160x over previous
import jax
import jax.numpy as jnp
from jax import lax
from jax.experimental import pallas as pl
from jax.experimental.pallas import tpu as pltpu

KQ = 8
CH = 512 // KQ


def kernel(x):
    m_per, n = x.shape
    half = m_per // 2
    quart = half // 2

    def body(x_ref, out_ref, stage_ref, xs, xr, zs, zr, ys, yr):
        my_x = lax.axis_index("x")
        my_y = lax.axis_index("y")
        my_z = lax.axis_index("z")
        p = my_z % 2
        xpeer = (1 - my_x, my_y, my_z)
        ypeer = (my_x, 1 - my_y, my_z)
        zpeer = (my_x, my_y, my_z + 1 - 2 * p)

        stage_ref[...] = x_ref[
            pl.ds(my_y * half + p * quart, quart), :
        ].astype(jnp.bfloat16)

        barrier_sem = pltpu.get_barrier_semaphore()
        for nbr in (xpeer, ypeer, zpeer):
            pl.semaphore_signal(
                barrier_sem, inc=1, device_id=nbr,
                device_id_type=pl.DeviceIdType.MESH,
            )
        pl.semaphore_wait(barrier_sem, 3)

        base_o = (1 - my_x) * m_per
        q_yp = base_o + my_y * half + p * quart
        q_y1p = base_o + my_y * half + (1 - p) * quart
        q_1yp = base_o + (1 - my_y) * half + p * quart

        dst_x = my_x * m_per + my_y * half + p * quart

        x_rdmas = []
        for k in range(KQ):
            r = pltpu.make_async_remote_copy(
                src_ref=stage_ref.at[pl.ds(k * CH, CH), :],
                dst_ref=out_ref.at[pl.ds(dst_x + k * CH, CH), :],
                send_sem=xs.at[k], recv_sem=xr.at[k],
                device_id=xpeer, device_id_type=pl.DeviceIdType.MESH,
            )
            r.start()
            x_rdmas.append(r)

        out_ref[pl.ds(my_x * m_per, m_per), :] = x_ref[...].astype(jnp.bfloat16)

        def fwd(rows, peer, send, recv):
            return pltpu.make_async_remote_copy(
                src_ref=out_ref.at[pl.ds(rows, CH), :],
                dst_ref=out_ref.at[pl.ds(rows, CH), :],
                send_sem=send, recv_sem=recv,
                device_id=peer, device_id_type=pl.DeviceIdType.MESH,
            )

        z_rdmas = [None] * (KQ + 4)
        y_rdmas = [None] * (KQ + 4)

        for k in range(KQ):
            x_rdmas[k].wait_recv()
            z_rdmas[k] = fwd(q_yp + k * CH, zpeer, zs.at[k], zr.at[k])
            z_rdmas[k].start()
            y_rdmas[k] = fwd(q_yp + k * CH, ypeer, ys.at[k], yr.at[k])
            y_rdmas[k].start()

        zr_waited = [False] * (KQ + 4)
        yr_waited = [False] * (KQ + 4)
        for j in range(4):
            pltpu.make_async_remote_copy(
                src_ref=out_ref.at[pl.ds(q_y1p + j * CH, CH), :],
                dst_ref=out_ref.at[pl.ds(q_y1p + j * CH, CH), :],
                send_sem=zs.at[0], recv_sem=zr.at[j],
                device_id=zpeer, device_id_type=pl.DeviceIdType.MESH,
            ).wait_recv()
            zr_waited[j] = True
            y_rdmas[KQ + j] = fwd(q_y1p + j * CH, ypeer, ys.at[KQ + j],
                                  yr.at[KQ + j])
            y_rdmas[KQ + j].start()
        for j in range(4):
            pltpu.make_async_remote_copy(
                src_ref=out_ref.at[pl.ds(q_1yp + (4 + j) * CH, CH), :],
                dst_ref=out_ref.at[pl.ds(q_1yp + (4 + j) * CH, CH), :],
                send_sem=ys.at[0], recv_sem=yr.at[4 + j],
                device_id=ypeer, device_id_type=pl.DeviceIdType.MESH,
            ).wait_recv()
            yr_waited[4 + j] = True
            z_rdmas[KQ + j] = fwd(q_1yp + (4 + j) * CH, zpeer, zs.at[KQ + j],
                                  zr.at[KQ + j])
            z_rdmas[KQ + j].start()

        for k in range(KQ):
            x_rdmas[k].wait_send()
        for j in range(KQ + 4):
            z_rdmas[j].wait_send()
            y_rdmas[j].wait_send()
        for j in range(KQ + 4):
            if not zr_waited[j]:
                pltpu.make_async_remote_copy(
                    src_ref=out_ref.at[pl.ds(base_o, CH), :],
                    dst_ref=out_ref.at[pl.ds(base_o, CH), :],
                    send_sem=zs.at[0], recv_sem=zr.at[j],
                    device_id=zpeer, device_id_type=pl.DeviceIdType.MESH,
                ).wait_recv()
            if not yr_waited[j]:
                pltpu.make_async_remote_copy(
                    src_ref=out_ref.at[pl.ds(base_o, CH), :],
                    dst_ref=out_ref.at[pl.ds(base_o, CH), :],
                    send_sem=ys.at[0], recv_sem=yr.at[j],
                    device_id=ypeer, device_id_type=pl.DeviceIdType.MESH,
                ).wait_recv()

    return pl.pallas_call(
        body,
        out_shape=jax.ShapeDtypeStruct((2 * m_per, n), jnp.bfloat16),
        in_specs=[pl.BlockSpec(memory_space=pltpu.VMEM)],
        out_specs=pl.BlockSpec(memory_space=pltpu.VMEM),
        scratch_shapes=[
            pltpu.VMEM((quart, n), jnp.bfloat16),
            pltpu.SemaphoreType.DMA((KQ,)),
            pltpu.SemaphoreType.DMA((KQ,)),
            pltpu.SemaphoreType.DMA((KQ + 4,)),
            pltpu.SemaphoreType.DMA((KQ + 4,)),
            pltpu.SemaphoreType.DMA((KQ + 4,)),
            pltpu.SemaphoreType.DMA((KQ + 4,)),
        ],
        compiler_params=pltpu.CompilerParams(collective_id=0),
    )(x)


# device time: 19985 ns/iter; 1.0559x vs baseline; 1.0393x over previous
import jax
import jax.numpy as jnp
from jax import lax
from jax.experimental import pallas as pl
from jax.experimental.pallas import tpu as pltpu

KQ = 8
CH = 512 // KQ
DX = 3
DY = 3
DZ = KQ - DX - DY


def kernel(x):
    m_per, n = x.shape
    half = m_per // 2
    quart = half // 2

    def body(x_ref, out_ref, stage_ref, stage2_ref, xs, xr, zs, zr, ys, yr):
        my_x = lax.axis_index("x")
        my_y = lax.axis_index("y")
        my_z = lax.axis_index("z")
        p = my_z % 2
        xpeer = (1 - my_x, my_y, my_z)
        ypeer = (my_x, 1 - my_y, my_z)
        zpeer = (my_x, my_y, my_z + 1 - 2 * p)

        barrier_sem = pltpu.get_barrier_semaphore()
        for nbr in (xpeer, ypeer, zpeer):
            pl.semaphore_signal(
                barrier_sem, inc=1, device_id=nbr,
                device_id_type=pl.DeviceIdType.MESH,
            )

        stage_ref[...] = x_ref[
            pl.ds(my_y * half + p * quart, quart), :
        ].astype(jnp.bfloat16)
        stage2_ref[...] = x_ref[
            pl.ds((1 - my_y) * half + (1 - p) * quart, DX * CH), :
        ].astype(jnp.bfloat16)

        pl.semaphore_wait(barrier_sem, 3)

        base_o = (1 - my_x) * m_per
        q_yp = base_o + my_y * half + p * quart
        q_y1p = base_o + my_y * half + (1 - p) * quart
        q_1yp = base_o + (1 - my_y) * half + p * quart

        dst_x = my_x * m_per + my_y * half + p * quart
        dst_x2 = my_x * m_per + (1 - my_y) * half + (1 - p) * quart

        x_rdmas = []
        for k in range(KQ):
            r = pltpu.make_async_remote_copy(
                src_ref=stage_ref.at[pl.ds(k * CH, CH), :],
                dst_ref=out_ref.at[pl.ds(dst_x + k * CH, CH), :],
                send_sem=xs.at[k], recv_sem=xr.at[k],
                device_id=xpeer, device_id_type=pl.DeviceIdType.MESH,
            )
            r.start()
            x_rdmas.append(r)
        for c in range(DX):
            r = pltpu.make_async_remote_copy(
                src_ref=stage2_ref.at[pl.ds(c * CH, CH), :],
                dst_ref=out_ref.at[pl.ds(dst_x2 + c * CH, CH), :],
                send_sem=xs.at[KQ + c], recv_sem=xr.at[KQ + c],
                device_id=xpeer, device_id_type=pl.DeviceIdType.MESH,
            )
            r.start()
            x_rdmas.append(r)

        def fwd(rows, peer, send, recv):
            return pltpu.make_async_remote_copy(
                src_ref=out_ref.at[pl.ds(rows, CH), :],
                dst_ref=out_ref.at[pl.ds(rows, CH), :],
                send_sem=send, recv_sem=recv,
                device_id=peer, device_id_type=pl.DeviceIdType.MESH,
            )

        recv_only = fwd

        z_rdmas = [None] * (KQ + DZ)
        y_rdmas = [None] * (KQ + DY)

        for k in range(KQ):
            x_rdmas[k].wait_recv()
            z_rdmas[k] = fwd(q_yp + k * CH, zpeer, zs.at[k], zr.at[k])
            z_rdmas[k].start()
            y_rdmas[k] = fwd(q_yp + k * CH, ypeer, ys.at[k], yr.at[k])
            y_rdmas[k].start()

        zr_waited = [False] * (KQ + DZ)
        yr_waited = [False] * (KQ + DY)

        for i in range(DY):
            j = DX + i
            recv_only(q_y1p + j * CH, zpeer, zs.at[0], zr.at[j]).wait_recv()
            zr_waited[j] = True
            y_rdmas[KQ + i] = fwd(q_y1p + j * CH, ypeer,
                                  ys.at[KQ + i], yr.at[KQ + i])
            y_rdmas[KQ + i].start()

        for i in range(DZ):
            j = DX + DY + i
            recv_only(q_1yp + j * CH, ypeer, ys.at[0], yr.at[j]).wait_recv()
            yr_waited[j] = True
            z_rdmas[KQ + i] = fwd(q_1yp + j * CH, zpeer,
                                  zs.at[KQ + i], zr.at[KQ + i])
            z_rdmas[KQ + i].start()

        out_ref[pl.ds(my_x * m_per, m_per), :] = x_ref[...].astype(jnp.bfloat16)

        for k in range(KQ + DX):
            x_rdmas[k].wait_send()
        for c in range(DX):
            x_rdmas[KQ + c].wait_recv()
        for j in range(KQ + DZ):
            z_rdmas[j].wait_send()
            if not zr_waited[j]:
                recv_only(base_o, zpeer, zs.at[0], zr.at[j]).wait_recv()
        for j in range(KQ + DY):
            y_rdmas[j].wait_send()
            if not yr_waited[j]:
                recv_only(base_o, ypeer, ys.at[0], yr.at[j]).wait_recv()

    return pl.pallas_call(
        body,
        out_shape=jax.ShapeDtypeStruct((2 * m_per, n), jnp.bfloat16),
        in_specs=[pl.BlockSpec(memory_space=pltpu.VMEM)],
        out_specs=pl.BlockSpec(memory_space=pltpu.VMEM),
        scratch_shapes=[
            pltpu.VMEM((quart, n), jnp.bfloat16),
            pltpu.VMEM((DX * CH, n), jnp.bfloat16),
            pltpu.SemaphoreType.DMA((KQ + DX,)),
            pltpu.SemaphoreType.DMA((KQ + DX,)),
            pltpu.SemaphoreType.DMA((KQ + DZ,)),
            pltpu.SemaphoreType.DMA((KQ + DZ,)),
            pltpu.SemaphoreType.DMA((KQ + DY,)),
            pltpu.SemaphoreType.DMA((KQ + DY,)),
        ],
        compiler_params=pltpu.CompilerParams(collective_id=0),
    )(x)
